# baseline (speedup 1.0000x reference)
import functools

import jax
import jax.numpy as jnp
from jax import lax
from jax.experimental import pallas as pl
from jax.experimental.pallas import tpu as pltpu
from jax.experimental.pallas import tpu_sc as plsc

_N = 10000
_E = 320000
_DIN = 128
_DOUT = 256

_NC = 2
_NS = 16
_NW = _NC * _NS
_CHUNK = 128
_NCHUNK = 80
_HALF = _NCHUNK // 2
_EPW = _CHUNK * _NCHUNK
_EPAD = _NW * _EPW
_RPAD = 10240
_RPT = _RPAD // _NS
_ZC = 128
_MBLK = 512

_mesh = plsc.VectorSubcoreMesh(core_axis_name="c", subcore_axis_name="s")


@functools.partial(
    pl.kernel,
    mesh=_mesh,
    out_type=jax.ShapeDtypeStruct((_NC, _RPAD, _DIN), jnp.float32),
    scratch_types=[
        pltpu.VMEM((_CHUNK, _DIN), jnp.float32),
        pltpu.VMEM((_CHUNK, _DIN), jnp.float32),
        pltpu.VMEM((_HALF, _CHUNK), jnp.int32),
        pltpu.VMEM((_HALF, _CHUNK), jnp.int32),
        pltpu.VMEM_SHARED((_RPAD, _DIN), jnp.float32),
        pltpu.SemaphoreType.DMA,
        pltpu.SemaphoreType.DMA,
    ],
)
def _sc_segsum(src_hbm, dst_hbm, x_hbm, zeros_hbm, out_hbm,
               rows0_v, rows1_v, src_v, dst_v, acc_sh, sem0, sem1):
    c = lax.axis_index("c")
    s = lax.axis_index("s")
    wid = c * _NS + s
    for k in range(_RPT // _ZC):
        pltpu.sync_copy(zeros_hbm, acc_sh.at[pl.ds(s * _RPT + k * _ZC, _ZC)])
    plsc.subcore_barrier()

    def body(g, carry):
        j0 = 2 * g
        j1 = 2 * g + 1
        h0 = pltpu.async_copy(x_hbm.at[src_v.at[j0]], rows0_v, sem0)
        h1 = pltpu.async_copy(x_hbm.at[src_v.at[j1]], rows1_v, sem1)
        h0.wait()
        pltpu.sync_copy(rows0_v, acc_sh.at[dst_v.at[j0]], add=True)
        h1.wait()
        pltpu.sync_copy(rows1_v, acc_sh.at[dst_v.at[j1]], add=True)
        return carry

    for h in range(_NCHUNK // _HALF):
        pltpu.sync_copy(src_hbm.at[wid, pl.ds(h * _HALF, _HALF)], src_v)
        pltpu.sync_copy(dst_hbm.at[wid, pl.ds(h * _HALF, _HALF)], dst_v)
        lax.fori_loop(0, _HALF // 2, body, 0)

    plsc.subcore_barrier()
    for k in range(_RPT // _ZC):
        r0 = s * _RPT + k * _ZC
        pltpu.sync_copy(acc_sh.at[pl.ds(r0, _ZC)], out_hbm.at[c, pl.ds(r0, _ZC)])


def _tc_body(a_ref, w_ref, b_ref, o_ref):
    blk = a_ref[0] + a_ref[1]
    y = jnp.dot(blk, w_ref[...], preferred_element_type=jnp.float32)
    o_ref[...] = jnp.maximum(y + b_ref[...], 0.0)


_tc_matmul = pl.pallas_call(
    _tc_body,
    grid=(_RPAD // _MBLK,),
    in_specs=[
        pl.BlockSpec((_NC, _MBLK, _DIN), lambda i: (0, i, 0)),
        pl.BlockSpec((_DIN, _DOUT), lambda i: (0, 0)),
        pl.BlockSpec((1, _DOUT), lambda i: (0, 0)),
    ],
    out_specs=pl.BlockSpec((_MBLK, _DOUT), lambda i: (i, 0)),
    out_shape=jax.ShapeDtypeStruct((_N, _DOUT), jnp.float32),
)


def kernel(x, edge_index, W, b):
    ei = edge_index.astype(jnp.int32)
    npad = _EPAD - _E
    src = jnp.concatenate([ei[0], jnp.zeros((npad,), jnp.int32)])
    dst = jnp.concatenate([ei[1], jnp.full((npad,), _N, jnp.int32)])
    src = src.reshape(_NW, _NCHUNK, _CHUNK)
    dst = dst.reshape(_NW, _NCHUNK, _CHUNK)
    zeros = jnp.zeros((_ZC, _DIN), jnp.float32)
    acc = _sc_segsum(src, dst, x, zeros)
    return _tc_matmul(acc, W, b.reshape(1, _DOUT))
